# Initial kernel scaffold; baseline (speedup 1.0000x reference)
import jax
import jax.numpy as jnp
from jax.experimental import pallas as pl


def kernel(hidden_states, key_cache, value_cache, Wq, Wk, Wv, Wo, rot_mat1, rot_mat2, ln1_w, ln2_w, Wg, Wu, Wd):
    raise NotImplementedError("write your pallas kernel here")


# speedup vs baseline: 3.9550x; 3.9550x over previous
import functools

import jax
import jax.numpy as jnp
import numpy as np
from jax.experimental import pallas as pl
from jax.experimental.pallas import tpu as pltpu

B = 1; Q = 8; KV = 4096; H = 32; HD = 128; D = 4096; FF = 11008
L = KV + Q
LP = 4224
NBLK = LP // HD
NUM_REMAIN = L - int(L * 0.9)
ROPE_BASE = 10000.0
INV_SQRT_HD = 1.0 / np.sqrt(HD).astype(np.float32)
NEG = float(jnp.finfo(jnp.float32).min)
F32 = jnp.float32

_DB = 256
_FB = 256


def _rot_half(x):
    rolled = jnp.roll(x, HD // 2, axis=-1)
    lane = jax.lax.broadcasted_iota(jnp.int32, x.shape, len(x.shape) - 1)
    return jnp.where(lane < HD // 2, -rolled, rolled)


def _mm(a, b, ca, cb):
    return jax.lax.dot_general(a, b, (((ca,), (cb,)), ((), ())),
                               preferred_element_type=F32)


def _rms(x, w):
    ms = jnp.mean(x * x, axis=-1, keepdims=True)
    return x * jax.lax.rsqrt(ms + 1e-6) * w


def _qkv_body(h_ref, w1_ref, wq_ref, wk_ref, wv_ref, q_ref, k_ref, v_ref):
    hn = _rms(h_ref[...], w1_ref[...])
    q_ref[...] = _mm(hn, wq_ref[...], 1, 1)
    k_ref[...] = _mm(hn, wk_ref[...], 1, 1)
    v_ref[...] = _mm(hn, wv_ref[...], 1, 1)


def _qkv_call(hid, w1, Wq, Wk, Wv):
    n = D // _DB
    return pl.pallas_call(
        _qkv_body,
        grid=(n,),
        in_specs=[
            pl.BlockSpec((Q, D), lambda i: (0, 0)),
            pl.BlockSpec((1, D), lambda i: (0, 0)),
            pl.BlockSpec((_DB, D), lambda i: (i, 0)),
            pl.BlockSpec((_DB, D), lambda i: (i, 0)),
            pl.BlockSpec((_DB, D), lambda i: (i, 0)),
        ],
        out_specs=[pl.BlockSpec((Q, _DB), lambda i: (0, i))] * 3,
        out_shape=[jax.ShapeDtypeStruct((Q, D), F32)] * 3,
    )(hid, w1, Wq, Wk, Wv)


def _hp_body(q_ref, k_ref, r1_ref, r2_ref, cos_ref, sin_ref,
             qr_ref, qh_ref, dn_ref, rn_ref):
    q = q_ref[...].reshape(Q, HD)
    k = k_ref[...].reshape(Q, HD)
    cos = cos_ref[...]
    sin = sin_ref[...]
    r1 = r1_ref[...].reshape(HD, HD)
    r2 = r2_ref[...].reshape(HD, HD)
    qr = q * cos + _rot_half(q) * sin
    kr = k * cos + _rot_half(k) * sin
    qi = _mm(jax.nn.silu(_mm(qr, r1, 1, 0)), r2, 1, 0)
    ki = _mm(jax.nn.silu(_mm(kr, r1, 1, 0)), r2, 1, 0)
    qs = jnp.sign(qi)
    ks = jnp.sign(ki)
    qr_ref[...] = qr.reshape(1, Q, HD)
    qh_ref[...] = qs.reshape(1, Q, HD)
    dn_ref[...] = _mm(qs, ks, 1, 1).reshape(1, Q, Q)
    rn_ref[...] = (_mm(qr, kr, 1, 1) * INV_SQRT_HD).reshape(1, Q, Q)


def _hp_call(qh, kh, r1, r2, cos_n, sin_n):
    return pl.pallas_call(
        _hp_body,
        grid=(H,),
        in_specs=[
            pl.BlockSpec((1, Q, HD), lambda i: (i, 0, 0)),
            pl.BlockSpec((1, Q, HD), lambda i: (i, 0, 0)),
            pl.BlockSpec((1, HD, HD), lambda i: (i, 0, 0)),
            pl.BlockSpec((1, HD, HD), lambda i: (i, 0, 0)),
            pl.BlockSpec((Q, HD), lambda i: (0, 0)),
            pl.BlockSpec((Q, HD), lambda i: (0, 0)),
        ],
        out_specs=[
            pl.BlockSpec((1, Q, HD), lambda i: (i, 0, 0)),
            pl.BlockSpec((1, Q, HD), lambda i: (i, 0, 0)),
            pl.BlockSpec((1, Q, Q), lambda i: (i, 0, 0)),
            pl.BlockSpec((1, Q, Q), lambda i: (i, 0, 0)),
        ],
        out_shape=[
            jax.ShapeDtypeStruct((H, Q, HD), F32),
            jax.ShapeDtypeStruct((H, Q, HD), F32),
            jax.ShapeDtypeStruct((H, Q, Q), F32),
            jax.ShapeDtypeStruct((H, Q, Q), F32),
        ],
    )(qh, kh, r1, r2, cos_n, sin_n)


def _score_body(kc_ref, r1_ref, r2_ref, cos_ref, sin_ref, qr_ref, qh_ref,
                d_ref, r_ref):
    k = kc_ref[...].reshape(KV, HD)
    kr = k * cos_ref[...] + _rot_half(k) * sin_ref[...]
    r1 = r1_ref[...].reshape(HD, HD)
    r2 = r2_ref[...].reshape(HD, HD)
    ki = _mm(jax.nn.silu(_mm(kr, r1, 1, 0)), r2, 1, 0)
    ks = jnp.sign(ki)
    qh = qh_ref[...].reshape(Q, HD)
    qr = qr_ref[...].reshape(Q, HD)
    d_ref[...] = _mm(qh, ks, 1, 1).reshape(1, Q, KV)
    r_ref[...] = (_mm(qr, kr, 1, 1) * INV_SQRT_HD).reshape(1, Q, KV)


def _score_call(kc, r1, r2, cos_c, sin_c, q_rope, q_hash):
    return pl.pallas_call(
        _score_body,
        grid=(H,),
        in_specs=[
            pl.BlockSpec((1, KV, HD), lambda i: (i, 0, 0)),
            pl.BlockSpec((1, HD, HD), lambda i: (i, 0, 0)),
            pl.BlockSpec((1, HD, HD), lambda i: (i, 0, 0)),
            pl.BlockSpec((KV, HD), lambda i: (0, 0)),
            pl.BlockSpec((KV, HD), lambda i: (0, 0)),
            pl.BlockSpec((1, Q, HD), lambda i: (i, 0, 0)),
            pl.BlockSpec((1, Q, HD), lambda i: (i, 0, 0)),
        ],
        out_specs=[
            pl.BlockSpec((1, Q, KV), lambda i: (i, 0, 0)),
            pl.BlockSpec((1, Q, KV), lambda i: (i, 0, 0)),
        ],
        out_shape=[
            jax.ShapeDtypeStruct((H, Q, KV), F32),
            jax.ShapeDtypeStruct((H, Q, KV), F32),
        ],
    )(kc, r1, r2, cos_c, sin_c, q_rope, q_hash)


def _att_body(d_ref, r_ref, vc_ref, vn_ref, o_ref):
    d = d_ref[...].reshape(Q, LP)
    rl = r_ref[...].reshape(Q, LP)
    lo = jnp.full((Q, 1), -128.0, F32)
    hi = jnp.full((Q, 1), 128.0, F32)
    for _ in range(9):
        mid = jnp.floor((lo + hi + 1.0) * 0.5)
        cnt = jnp.sum(jnp.where(d >= mid, 1.0, 0.0), axis=-1, keepdims=True)
        ge = cnt >= float(NUM_REMAIN)
        lo = jnp.where(ge, mid, lo)
        hi = jnp.where(ge, hi, mid - 1.0)
    thr = lo
    gcnt = jnp.sum(jnp.where(d > thr, 1.0, 0.0), axis=-1, keepdims=True)
    rrem = float(NUM_REMAIN) - gcnt
    e = jnp.where(d == thr, 1.0, 0.0)
    ii = jax.lax.broadcasted_iota(jnp.int32, (HD, HD), 0)
    jj = jax.lax.broadcasted_iota(jnp.int32, (HD, HD), 1)
    tri = jnp.where(ii <= jj, 1.0, 0.0)
    carry = jnp.zeros((Q, 1), F32)
    parts = []
    for b in range(NBLK):
        cblk = _mm(e[:, b * HD:(b + 1) * HD], tri, 1, 0) + carry
        parts.append(cblk)
        carry = cblk[:, HD - 1:HD]
    cum = jnp.concatenate(parts, axis=-1)
    sel = (d > thr) | ((e > 0.5) & (cum <= rrem))
    masked = jnp.where(sel, rl, NEG)
    m = jnp.max(masked, axis=-1, keepdims=True)
    p = jnp.where(sel, jnp.exp(rl - m), 0.0)
    denom = jnp.sum(p, axis=-1, keepdims=True)
    att = _mm(p[:, :KV], vc_ref[...].reshape(KV, HD), 1, 0)
    att = att + _mm(p[:, KV:LP], vn_ref[...].reshape(HD, HD), 1, 0)
    o_ref[...] = (att / denom).reshape(1, Q, HD)


def _att_call(draft_p, real_p, vc, v_new_pad):
    return pl.pallas_call(
        _att_body,
        grid=(H,),
        in_specs=[
            pl.BlockSpec((1, Q, LP), lambda i: (i, 0, 0)),
            pl.BlockSpec((1, Q, LP), lambda i: (i, 0, 0)),
            pl.BlockSpec((1, KV, HD), lambda i: (i, 0, 0)),
            pl.BlockSpec((1, HD, HD), lambda i: (i, 0, 0)),
        ],
        out_specs=pl.BlockSpec((1, Q, HD), lambda i: (i, 0, 0)),
        out_shape=jax.ShapeDtypeStruct((H, Q, HD), F32),
    )(draft_p, real_p, vc, v_new_pad)


def _op_body(a_ref, hid_ref, wo_ref, o_ref):
    o_ref[...] = _mm(a_ref[...], wo_ref[...], 1, 1) + hid_ref[...]


def _op_call(attn_f, hid, Wo):
    n = D // _DB
    return pl.pallas_call(
        _op_body,
        grid=(n,),
        in_specs=[
            pl.BlockSpec((Q, D), lambda i: (0, 0)),
            pl.BlockSpec((Q, _DB), lambda i: (0, i)),
            pl.BlockSpec((_DB, D), lambda i: (i, 0)),
        ],
        out_specs=pl.BlockSpec((Q, _DB), lambda i: (0, i)),
        out_shape=jax.ShapeDtypeStruct((Q, D), F32),
    )(attn_f, hid, Wo)


def _mlp_body(h_ref, w2_ref, wg_ref, wu_ref, wd_ref, o_ref):
    i = pl.program_id(0)
    h = h_ref[...]
    hn = _rms(h, w2_ref[...])
    g = jax.nn.silu(_mm(hn, wg_ref[...], 1, 1))
    u = _mm(hn, wu_ref[...], 1, 1)
    part = _mm(g * u, wd_ref[...], 1, 1)

    @pl.when(i == 0)
    def _():
        o_ref[...] = h + part

    @pl.when(i > 0)
    def _():
        o_ref[...] += part


def _mlp_call(h_res, w2, Wg, Wu, Wd):
    n = FF // _FB
    return pl.pallas_call(
        _mlp_body,
        grid=(n,),
        in_specs=[
            pl.BlockSpec((Q, D), lambda i: (0, 0)),
            pl.BlockSpec((1, D), lambda i: (0, 0)),
            pl.BlockSpec((_FB, D), lambda i: (i, 0)),
            pl.BlockSpec((_FB, D), lambda i: (i, 0)),
            pl.BlockSpec((D, _FB), lambda i: (0, i)),
        ],
        out_specs=pl.BlockSpec((Q, D), lambda i: (0, 0)),
        out_shape=jax.ShapeDtypeStruct((Q, D), F32),
    )(h_res, w2, Wg, Wu, Wd)


def kernel(hidden_states, key_cache, value_cache, Wq, Wk, Wv, Wo,
           rot_mat1, rot_mat2, ln1_w, ln2_w, Wg, Wu, Wd):
    hid = hidden_states.reshape(Q, D)
    kc = key_cache.reshape(H, KV, HD)
    vc = value_cache.reshape(H, KV, HD)
    r1 = rot_mat1.reshape(H, HD, HD)
    r2 = rot_mat2.reshape(H, HD, HD)
    w1 = ln1_w.reshape(1, D)
    w2 = ln2_w.reshape(1, D)

    inv_freq = 1.0 / (ROPE_BASE ** (jnp.arange(0, HD, 2, dtype=F32) / HD))
    t = jnp.arange(L, dtype=F32)
    freqs = jnp.outer(t, inv_freq)
    emb = jnp.concatenate([freqs, freqs], axis=-1)
    cos = jnp.cos(emb)
    sin = jnp.sin(emb)
    cos_c, cos_n = cos[:KV], cos[KV:]
    sin_c, sin_n = sin[:KV], sin[KV:]

    q_f, k_f, v_f = _qkv_call(hid, w1, Wq, Wk, Wv)
    qh = q_f.reshape(Q, H, HD).transpose(1, 0, 2)
    kh = k_f.reshape(Q, H, HD).transpose(1, 0, 2)
    vh = v_f.reshape(Q, H, HD).transpose(1, 0, 2)
    v_new_pad = jnp.pad(vh, ((0, 0), (0, HD - Q), (0, 0)))

    q_rope, q_hash, draft_new, real_new = _hp_call(qh, kh, r1, r2, cos_n, sin_n)
    draft_c, real_c = _score_call(kc, r1, r2, cos_c, sin_c, q_rope, q_hash)

    draft_p = jnp.concatenate(
        [draft_c, draft_new, jnp.full((H, Q, LP - L), -1000.0, F32)], axis=-1)
    real_p = jnp.concatenate(
        [real_c, real_new, jnp.zeros((H, Q, LP - L), F32)], axis=-1)

    attn = _att_call(draft_p, real_p, vc, v_new_pad)
    attn_f = attn.transpose(1, 0, 2).reshape(Q, D)
    h_res = _op_call(attn_f, hid, Wo)
    out = _mlp_call(h_res, w2, Wg, Wu, Wd)
    return out.reshape(B, Q, D)


# speedup vs baseline: 5.0162x; 1.2683x over previous
import functools

import jax
import jax.numpy as jnp
import numpy as np
from jax.experimental import pallas as pl
from jax.experimental.pallas import tpu as pltpu

B = 1; Q = 8; KV = 4096; H = 32; HD = 128; D = 4096; FF = 11008
L = KV + Q
LP = 4224
NBLK = LP // HD
NUM_REMAIN = L - int(L * 0.9)
ROPE_BASE = 10000.0
INV_SQRT_HD = 1.0 / np.sqrt(HD).astype(np.float32)
NEG = float(jnp.finfo(jnp.float32).min)
F32 = jnp.float32

_DB = 256
_FB = 256


def _rot_half(x):
    rolled = jnp.roll(x, HD // 2, axis=-1)
    lane = jax.lax.broadcasted_iota(jnp.int32, x.shape, len(x.shape) - 1)
    return jnp.where(lane < HD // 2, -rolled, rolled)


def _mm(a, b, ca, cb):
    return jax.lax.dot_general(a, b, (((ca,), (cb,)), ((), ())),
                               preferred_element_type=F32)


def _rms(x, w):
    ms = jnp.mean(x * x, axis=-1, keepdims=True)
    return x * jax.lax.rsqrt(ms + 1e-6) * w


def _qkv_body(h_ref, w1_ref, wq_ref, wk_ref, wv_ref, q_ref, k_ref, v_ref):
    hn = _rms(h_ref[...], w1_ref[...])
    q_ref[...] = _mm(hn, wq_ref[...], 1, 1)
    k_ref[...] = _mm(hn, wk_ref[...], 1, 1)
    v_ref[...] = _mm(hn, wv_ref[...], 1, 1)


def _qkv_call(hid, w1, Wq, Wk, Wv):
    n = D // _DB
    return pl.pallas_call(
        _qkv_body,
        grid=(n,),
        in_specs=[
            pl.BlockSpec((Q, D), lambda i: (0, 0)),
            pl.BlockSpec((1, D), lambda i: (0, 0)),
            pl.BlockSpec((_DB, D), lambda i: (i, 0)),
            pl.BlockSpec((_DB, D), lambda i: (i, 0)),
            pl.BlockSpec((_DB, D), lambda i: (i, 0)),
        ],
        out_specs=[pl.BlockSpec((Q, _DB), lambda i: (0, i))] * 3,
        out_shape=[jax.ShapeDtypeStruct((Q, D), F32)] * 3,
    )(hid, w1, Wq, Wk, Wv)


def _hp_body(q_ref, k_ref, r1_ref, r2_ref, cos_ref, sin_ref,
             qr_ref, qh_ref, dn_ref, rn_ref):
    q = q_ref[...].reshape(Q, HD)
    k = k_ref[...].reshape(Q, HD)
    cos = cos_ref[...]
    sin = sin_ref[...]
    r1 = r1_ref[...].reshape(HD, HD)
    r2 = r2_ref[...].reshape(HD, HD)
    qr = q * cos + _rot_half(q) * sin
    kr = k * cos + _rot_half(k) * sin
    qi = _mm(jax.nn.silu(_mm(qr, r1, 1, 0)), r2, 1, 0)
    ki = _mm(jax.nn.silu(_mm(kr, r1, 1, 0)), r2, 1, 0)
    qs = jnp.sign(qi)
    ks = jnp.sign(ki)
    qr_ref[...] = qr.reshape(1, Q, HD)
    qh_ref[...] = qs.reshape(1, Q, HD)
    dn_ref[...] = _mm(qs, ks, 1, 1).reshape(1, Q, Q)
    rn_ref[...] = (_mm(qr, kr, 1, 1) * INV_SQRT_HD).reshape(1, Q, Q)


def _hp_call(qh, kh, r1, r2, cos_n, sin_n):
    return pl.pallas_call(
        _hp_body,
        grid=(H,),
        in_specs=[
            pl.BlockSpec((1, Q, HD), lambda i: (i, 0, 0)),
            pl.BlockSpec((1, Q, HD), lambda i: (i, 0, 0)),
            pl.BlockSpec((1, HD, HD), lambda i: (i, 0, 0)),
            pl.BlockSpec((1, HD, HD), lambda i: (i, 0, 0)),
            pl.BlockSpec((Q, HD), lambda i: (0, 0)),
            pl.BlockSpec((Q, HD), lambda i: (0, 0)),
        ],
        out_specs=[
            pl.BlockSpec((1, Q, HD), lambda i: (i, 0, 0)),
            pl.BlockSpec((1, Q, HD), lambda i: (i, 0, 0)),
            pl.BlockSpec((1, Q, Q), lambda i: (i, 0, 0)),
            pl.BlockSpec((1, Q, Q), lambda i: (i, 0, 0)),
        ],
        out_shape=[
            jax.ShapeDtypeStruct((H, Q, HD), F32),
            jax.ShapeDtypeStruct((H, Q, HD), F32),
            jax.ShapeDtypeStruct((H, Q, Q), F32),
            jax.ShapeDtypeStruct((H, Q, Q), F32),
        ],
    )(qh, kh, r1, r2, cos_n, sin_n)


def _score_body(kc_ref, r1_ref, r2_ref, cos_ref, sin_ref, qr_ref, qh_ref,
                d_ref, r_ref):
    k = kc_ref[...].reshape(KV, HD)
    kr = k * cos_ref[...] + _rot_half(k) * sin_ref[...]
    r1 = r1_ref[...].reshape(HD, HD)
    r2 = r2_ref[...].reshape(HD, HD)
    ki = _mm(jax.nn.silu(_mm(kr, r1, 1, 0)), r2, 1, 0)
    ks = jnp.sign(ki)
    qh = qh_ref[...].reshape(Q, HD)
    qr = qr_ref[...].reshape(Q, HD)
    d_ref[...] = _mm(qh, ks, 1, 1).reshape(1, Q, KV)
    r_ref[...] = (_mm(qr, kr, 1, 1) * INV_SQRT_HD).reshape(1, Q, KV)


def _score_call(kc, r1, r2, cos_c, sin_c, q_rope, q_hash):
    return pl.pallas_call(
        _score_body,
        grid=(H,),
        in_specs=[
            pl.BlockSpec((1, KV, HD), lambda i: (i, 0, 0)),
            pl.BlockSpec((1, HD, HD), lambda i: (i, 0, 0)),
            pl.BlockSpec((1, HD, HD), lambda i: (i, 0, 0)),
            pl.BlockSpec((KV, HD), lambda i: (0, 0)),
            pl.BlockSpec((KV, HD), lambda i: (0, 0)),
            pl.BlockSpec((1, Q, HD), lambda i: (i, 0, 0)),
            pl.BlockSpec((1, Q, HD), lambda i: (i, 0, 0)),
        ],
        out_specs=[
            pl.BlockSpec((1, Q, KV), lambda i: (i, 0, 0)),
            pl.BlockSpec((1, Q, KV), lambda i: (i, 0, 0)),
        ],
        out_shape=[
            jax.ShapeDtypeStruct((H, Q, KV), F32),
            jax.ShapeDtypeStruct((H, Q, KV), F32),
        ],
    )(kc, r1, r2, cos_c, sin_c, q_rope, q_hash)


_AH = 4
_AR = _AH * Q


def _att_body(dc_ref, dn_ref, rc_ref, rn_ref, vc_ref, vn_ref, o_ref):
    dc = dc_ref[...].reshape(_AR, KV)
    dn = dn_ref[...].reshape(_AR, Q)
    rc = rc_ref[...].reshape(_AR, KV)
    rn = rn_ref[...].reshape(_AR, Q)
    d = jnp.concatenate(
        [dc, dn, jnp.full((_AR, LP - L), -1000.0, F32)], axis=-1)
    rl = jnp.concatenate([rc, rn, jnp.zeros((_AR, LP - L), F32)], axis=-1)
    lo = jnp.full((_AR, 1), -128.0, F32)
    hi = jnp.full((_AR, 1), 128.0, F32)
    for _ in range(9):
        mid = jnp.floor((lo + hi + 1.0) * 0.5)
        cnt = jnp.sum(jnp.where(d >= mid, 1.0, 0.0), axis=-1, keepdims=True)
        ge = cnt >= float(NUM_REMAIN)
        lo = jnp.where(ge, mid, lo)
        hi = jnp.where(ge, hi, mid - 1.0)
    thr = lo
    gcnt = jnp.sum(jnp.where(d > thr, 1.0, 0.0), axis=-1, keepdims=True)
    rrem = float(NUM_REMAIN) - gcnt
    e = jnp.where(d == thr, 1.0, 0.0)
    ii = jax.lax.broadcasted_iota(jnp.int32, (HD, HD), 0)
    jj = jax.lax.broadcasted_iota(jnp.int32, (HD, HD), 1)
    tri = jnp.where(ii <= jj, 1.0, 0.0)
    cblks = [_mm(e[:, b * HD:(b + 1) * HD], tri, 1, 0) for b in range(NBLK)]
    lasts = jnp.concatenate([c[:, HD - 1:HD] for c in cblks], axis=-1)
    i3 = jax.lax.broadcasted_iota(jnp.int32, (NBLK, NBLK), 0)
    j3 = jax.lax.broadcasted_iota(jnp.int32, (NBLK, NBLK), 1)
    tri3 = jnp.where(i3 < j3, 1.0, 0.0)
    base = _mm(lasts, tri3, 1, 0)
    cum = jnp.concatenate(
        [cblks[b] + base[:, b:b + 1] for b in range(NBLK)], axis=-1)
    sel = (d > thr) | ((e > 0.5) & (cum <= rrem))
    masked = jnp.where(sel, rl, NEG)
    m = jnp.max(masked, axis=-1, keepdims=True)
    p = jnp.where(sel, jnp.exp(rl - m), 0.0)
    denom = jnp.sum(p, axis=-1, keepdims=True)
    vc = vc_ref[...]
    vn = vn_ref[...]
    outs = []
    for hh in range(_AH):
        ph = p[hh * Q:(hh + 1) * Q]
        att = _mm(ph[:, :KV], vc[hh], 1, 0) + _mm(ph[:, KV:LP], vn[hh], 1, 0)
        outs.append(att / denom[hh * Q:(hh + 1) * Q])
    o_ref[...] = jnp.concatenate(outs, axis=0).reshape(_AH, Q, HD)


def _att_call(draft_c, draft_new, real_c, real_new, vc, v_new_pad):
    return pl.pallas_call(
        _att_body,
        grid=(H // _AH,),
        in_specs=[
            pl.BlockSpec((_AH, Q, KV), lambda i: (i, 0, 0)),
            pl.BlockSpec((_AH, Q, Q), lambda i: (i, 0, 0)),
            pl.BlockSpec((_AH, Q, KV), lambda i: (i, 0, 0)),
            pl.BlockSpec((_AH, Q, Q), lambda i: (i, 0, 0)),
            pl.BlockSpec((_AH, KV, HD), lambda i: (i, 0, 0)),
            pl.BlockSpec((_AH, HD, HD), lambda i: (i, 0, 0)),
        ],
        out_specs=pl.BlockSpec((_AH, Q, HD), lambda i: (i, 0, 0)),
        out_shape=jax.ShapeDtypeStruct((H, Q, HD), F32),
    )(draft_c, draft_new, real_c, real_new, vc, v_new_pad)


def _op_body(a_ref, hid_ref, wo_ref, o_ref):
    o_ref[...] = _mm(a_ref[...], wo_ref[...], 1, 1) + hid_ref[...]


def _op_call(attn_f, hid, Wo):
    n = D // _DB
    return pl.pallas_call(
        _op_body,
        grid=(n,),
        in_specs=[
            pl.BlockSpec((Q, D), lambda i: (0, 0)),
            pl.BlockSpec((Q, _DB), lambda i: (0, i)),
            pl.BlockSpec((_DB, D), lambda i: (i, 0)),
        ],
        out_specs=pl.BlockSpec((Q, _DB), lambda i: (0, i)),
        out_shape=jax.ShapeDtypeStruct((Q, D), F32),
    )(attn_f, hid, Wo)


def _mlp_body(h_ref, w2_ref, wg_ref, wu_ref, wd_ref, o_ref):
    i = pl.program_id(0)
    h = h_ref[...]
    hn = _rms(h, w2_ref[...])
    g = jax.nn.silu(_mm(hn, wg_ref[...], 1, 1))
    u = _mm(hn, wu_ref[...], 1, 1)
    part = _mm(g * u, wd_ref[...], 1, 1)

    @pl.when(i == 0)
    def _():
        o_ref[...] = h + part

    @pl.when(i > 0)
    def _():
        o_ref[...] += part


def _mlp_call(h_res, w2, Wg, Wu, Wd):
    n = FF // _FB
    return pl.pallas_call(
        _mlp_body,
        grid=(n,),
        in_specs=[
            pl.BlockSpec((Q, D), lambda i: (0, 0)),
            pl.BlockSpec((1, D), lambda i: (0, 0)),
            pl.BlockSpec((_FB, D), lambda i: (i, 0)),
            pl.BlockSpec((_FB, D), lambda i: (i, 0)),
            pl.BlockSpec((D, _FB), lambda i: (0, i)),
        ],
        out_specs=pl.BlockSpec((Q, D), lambda i: (0, 0)),
        out_shape=jax.ShapeDtypeStruct((Q, D), F32),
    )(h_res, w2, Wg, Wu, Wd)


def kernel(hidden_states, key_cache, value_cache, Wq, Wk, Wv, Wo,
           rot_mat1, rot_mat2, ln1_w, ln2_w, Wg, Wu, Wd):
    hid = hidden_states.reshape(Q, D)
    kc = key_cache.reshape(H, KV, HD)
    vc = value_cache.reshape(H, KV, HD)
    r1 = rot_mat1.reshape(H, HD, HD)
    r2 = rot_mat2.reshape(H, HD, HD)
    w1 = ln1_w.reshape(1, D)
    w2 = ln2_w.reshape(1, D)

    inv_freq = 1.0 / (ROPE_BASE ** (jnp.arange(0, HD, 2, dtype=F32) / HD))
    t = jnp.arange(L, dtype=F32)
    freqs = jnp.outer(t, inv_freq)
    emb = jnp.concatenate([freqs, freqs], axis=-1)
    cos = jnp.cos(emb)
    sin = jnp.sin(emb)
    cos_c, cos_n = cos[:KV], cos[KV:]
    sin_c, sin_n = sin[:KV], sin[KV:]

    q_f, k_f, v_f = _qkv_call(hid, w1, Wq, Wk, Wv)
    qh = q_f.reshape(Q, H, HD).transpose(1, 0, 2)
    kh = k_f.reshape(Q, H, HD).transpose(1, 0, 2)
    vh = v_f.reshape(Q, H, HD).transpose(1, 0, 2)
    v_new_pad = jnp.pad(vh, ((0, 0), (0, HD - Q), (0, 0)))

    q_rope, q_hash, draft_new, real_new = _hp_call(qh, kh, r1, r2, cos_n, sin_n)
    draft_c, real_c = _score_call(kc, r1, r2, cos_c, sin_c, q_rope, q_hash)

    attn = _att_call(draft_c, draft_new, real_c, real_new, vc, v_new_pad)
    attn_f = attn.transpose(1, 0, 2).reshape(Q, D)
    h_res = _op_call(attn_f, hid, Wo)
    out = _mlp_call(h_res, w2, Wg, Wu, Wd)
    return out.reshape(B, Q, D)
